# baseline (speedup 1.0000x reference)
import functools

import jax
import jax.numpy as jnp
import numpy as np
from jax import lax
from jax.experimental import pallas as pl
from jax.experimental.pallas import tpu as pltpu
from jax.experimental.pallas import tpu_sc as plsc

B = 4096
D = 128
NUM_SC_TABLES = 3
NC = 2
NS = 16
NW = NC * NS
SEG = B // NW
BB = 2048
HEX_BITS = 16
NUM_VOLATILE_FIELDS = 9
HEX_FEATS = NUM_VOLATILE_FIELDS * HEX_BITS

_P_NP = np.zeros((NUM_VOLATILE_FIELDS, HEX_FEATS), np.float32)
for _f in range(NUM_VOLATILE_FIELDS):
    for _k in range(HEX_BITS):
        _P_NP[_f, HEX_BITS * _f + _k] = 2.0 ** (-_k)


def _sc_gather(tables, idxs):
    mesh = plsc.VectorSubcoreMesh(core_axis_name="c", subcore_axis_name="s")
    n = NUM_SC_TABLES

    @functools.partial(
        pl.kernel,
        out_type=jax.ShapeDtypeStruct((n * B, D), jnp.float32),
        mesh=mesh,
        scratch_types=(
            [pltpu.VMEM((SEG,), jnp.int32) for _ in range(n)]
            + [pltpu.VMEM((SEG, D), jnp.float32) for _ in range(2)]
            + [pltpu.SemaphoreType.DMA for _ in range(n + 4)]
        ),
    )
    def gather_kernel(t0, t1, t2, i0, i1, i2, out_hbm, *scratch):
        tabs = (t0, t1, t2)
        idx_hbm = (i0, i1, i2)
        ib = scratch[:n]
        rb = scratch[n:n + 2]
        sis = scratch[n + 2:2 * n + 2]
        sgs = scratch[2 * n + 2:2 * n + 4]
        sws = scratch[2 * n + 4:2 * n + 6]
        wid = lax.axis_index("s") * NC + lax.axis_index("c")
        base = wid * SEG

        icp = [
            pltpu.async_copy(idx_hbm[k].at[pl.ds(base, SEG)], ib[k], sis[k])
            for k in range(n)
        ]
        gcp = [None] * n
        wcp = [None] * n
        for k in range(n):
            p = k % 2
            if k >= 2:
                wcp[k - 2].wait()
            icp[k].wait()
            gcp[k] = pltpu.async_copy(tabs[k].at[ib[k]], rb[p], sgs[p])
            if k >= 1:
                gcp[k - 1].wait()
                wcp[k - 1] = pltpu.async_copy(
                    rb[(k - 1) % 2],
                    out_hbm.at[pl.ds((k - 1) * B + base, SEG)],
                    sws[(k - 1) % 2])
        gcp[n - 1].wait()
        wcp[n - 1] = pltpu.async_copy(
            rb[(n - 1) % 2],
            out_hbm.at[pl.ds((n - 1) * B + base, SEG)],
            sws[(n - 1) % 2])
        wcp[n - 2].wait()
        wcp[n - 1].wait()

    return gather_kernel(*tables, *idxs)


def _fused_body(rows_ref, mv_ref, sd_ref, vol_ref, sp_ref, act_ref, side_ref,
                p_ref, whex_ref, wout_ref, b_ref, o_ref):
    mv = mv_ref[...]
    na = act_ref.shape[0]
    cols = lax.broadcasted_iota(jnp.int32, (BB, na), 1)
    counts = (
        (mv[:, 0:1] == cols).astype(jnp.bfloat16)
        + (mv[:, 1:2] == cols).astype(jnp.bfloat16)
        + (mv[:, 2:3] == cols).astype(jnp.bfloat16)
        + (mv[:, 3:4] == cols).astype(jnp.bfloat16)
    )
    mv_sum = jnp.dot(counts, act_ref[...],
                     preferred_element_type=jnp.float32)
    acc = jnp.maximum(mv_sum * 0.25, 0.0)
    sd = sd_ref[...]
    acc += jnp.maximum(
        jnp.where(sd == 0, side_ref[0:1, :], side_ref[1:2, :]), 0.0)
    del p_ref
    v = vol_ref[...]
    k16 = lax.broadcasted_iota(jnp.int32, (1, HEX_BITS), 1)
    bits = jnp.concatenate(
        [jnp.right_shift(v[:, f : f + 1], k16) & 1
         for f in range(NUM_VOLATILE_FIELDS)],
        axis=1,
    ).astype(jnp.float32)
    acc += jnp.dot(bits, whex_ref[...], preferred_element_type=jnp.float32)
    g = rows_ref[...]
    acc += jnp.maximum(g[0], 0.0) + jnp.maximum(g[1], 0.0)
    acc += jnp.maximum(g[2], 0.0)
    out = jnp.dot(acc, wout_ref[...], preferred_element_type=jnp.float32)
    out = jnp.maximum(out + b_ref[...], 0.0)
    o_ref[...] = jnp.where(sp_ref[...] != 0, out, 0.0)


def _tc_fused(rows3, mv, sd, vol, sp, actions, side_table, pmat, w_hex, w_out,
              b2):
    n_blocks = B // BB
    na = actions.shape[0]
    return pl.pallas_call(
        _fused_body,
        grid=(n_blocks,),
        in_specs=[
            pl.BlockSpec((NUM_SC_TABLES, BB, D), lambda i: (0, i, 0)),
            pl.BlockSpec((BB, 4), lambda i: (i, 0)),
            pl.BlockSpec((BB, 1), lambda i: (i, 0)),
            pl.BlockSpec((BB, NUM_VOLATILE_FIELDS), lambda i: (i, 0)),
            pl.BlockSpec((BB, 1), lambda i: (i, 0)),
            pl.BlockSpec((na, D), lambda i: (0, 0)),
            pl.BlockSpec((2, D), lambda i: (0, 0)),
            pl.BlockSpec((NUM_VOLATILE_FIELDS, HEX_FEATS), lambda i: (0, 0)),
            pl.BlockSpec((HEX_FEATS, D), lambda i: (0, 0)),
            pl.BlockSpec((D, D), lambda i: (0, 0)),
            pl.BlockSpec((1, D), lambda i: (0, 0)),
        ],
        out_specs=pl.BlockSpec((BB, D), lambda i: (i, 0)),
        out_shape=jax.ShapeDtypeStruct((B, D), jnp.float32),
    )(rows3, mv, sd, vol, sp, actions, side_table, pmat, w_hex, w_out, b2)


def kernel(species_idx, ability_idx, item_idx, side_idx, move_ids, volatiles,
           species_table, abilities_table, items_table, actions_table,
           side_table, W_hex, W_out, b_out):
    sp = species_idx.astype(jnp.int32)
    rows = _sc_gather(
        (species_table, abilities_table, items_table),
        (sp, ability_idx.astype(jnp.int32), item_idx.astype(jnp.int32)))
    rows3 = rows.reshape(NUM_SC_TABLES, B, D)
    actions_pad = jnp.zeros((1024, D), jnp.bfloat16).at[
        :actions_table.shape[0]].set(actions_table.astype(jnp.bfloat16))
    return _tc_fused(rows3, move_ids.astype(jnp.int32),
                     side_idx.astype(jnp.int32).reshape(B, 1),
                     volatiles.astype(jnp.int32), sp.reshape(B, 1),
                     actions_pad, side_table, jnp.asarray(_P_NP), W_hex,
                     W_out, b_out.reshape(1, D))


# speedup vs baseline: 1.0253x; 1.0253x over previous
import functools

import jax
import jax.numpy as jnp
import numpy as np
from jax import lax
from jax.experimental import pallas as pl
from jax.experimental.pallas import tpu as pltpu
from jax.experimental.pallas import tpu_sc as plsc

B = 4096
D = 128
NUM_SC_TABLES = 3
NC = 2
NS = 16
NW = NC * NS
SEG = B // NW
BB = 1024
HEX_BITS = 16
NUM_VOLATILE_FIELDS = 9
HEX_FEATS = NUM_VOLATILE_FIELDS * HEX_BITS

_P_NP = np.zeros((NUM_VOLATILE_FIELDS, HEX_FEATS), np.float32)
for _f in range(NUM_VOLATILE_FIELDS):
    for _k in range(HEX_BITS):
        _P_NP[_f, HEX_BITS * _f + _k] = 2.0 ** (-_k)


def _sc_gather(tables, idxs):
    mesh = plsc.VectorSubcoreMesh(core_axis_name="c", subcore_axis_name="s")
    n = NUM_SC_TABLES

    @functools.partial(
        pl.kernel,
        out_type=jax.ShapeDtypeStruct((n * B, D), jnp.float32),
        mesh=mesh,
        scratch_types=(
            [pltpu.VMEM((SEG,), jnp.int32) for _ in range(n)]
            + [pltpu.VMEM((SEG, D), jnp.float32) for _ in range(n)]
            + [pltpu.SemaphoreType.DMA for _ in range(3 * n)]
        ),
    )
    def gather_kernel(t0, t1, t2, i0, i1, i2, out_hbm, *scratch):
        tabs = (t0, t1, t2)
        idx_hbm = (i0, i1, i2)
        ib = scratch[:n]
        rb = scratch[n:2 * n]
        sis = scratch[2 * n:3 * n]
        sgs = scratch[3 * n:4 * n]
        sws = scratch[4 * n:5 * n]
        wid = lax.axis_index("s") * NC + lax.axis_index("c")
        base = wid * SEG

        icp = [
            pltpu.async_copy(idx_hbm[k].at[pl.ds(base, SEG)], ib[k], sis[k])
            for k in range(n)
        ]
        gcp = [None] * n
        wcp = [None] * n
        for k in range(n):
            icp[k].wait()
            gcp[k] = pltpu.async_copy(tabs[k].at[ib[k]], rb[k], sgs[k])
        for k in range(n):
            gcp[k].wait()
            wcp[k] = pltpu.async_copy(
                rb[k], out_hbm.at[pl.ds(k * B + base, SEG)], sws[k])
        for k in range(n):
            wcp[k].wait()

    return gather_kernel(*tables, *idxs)


def _fused_body(rows_ref, mv_ref, sd_ref, vol_ref, sp_ref, act_ref, side_ref,
                p_ref, whex_ref, wout_ref, b_ref, o_ref):
    mv = mv_ref[...]
    na = act_ref.shape[0]
    cols = lax.broadcasted_iota(jnp.int32, (BB, na), 1)
    counts = (
        (mv[:, 0:1] == cols).astype(jnp.bfloat16)
        + (mv[:, 1:2] == cols).astype(jnp.bfloat16)
        + (mv[:, 2:3] == cols).astype(jnp.bfloat16)
        + (mv[:, 3:4] == cols).astype(jnp.bfloat16)
    )
    mv_sum = jnp.dot(counts, act_ref[...],
                     preferred_element_type=jnp.float32)
    acc = jnp.maximum(mv_sum * 0.25, 0.0)
    sd = sd_ref[...]
    acc += jnp.maximum(
        jnp.where(sd == 0, side_ref[0:1, :], side_ref[1:2, :]), 0.0)
    del p_ref
    v = vol_ref[...]
    k16 = lax.broadcasted_iota(jnp.int32, (1, HEX_BITS), 1)
    bits = jnp.concatenate(
        [jnp.right_shift(v[:, f : f + 1], k16) & 1
         for f in range(NUM_VOLATILE_FIELDS)],
        axis=1,
    ).astype(jnp.float32)
    acc += jnp.dot(bits, whex_ref[...], preferred_element_type=jnp.float32)
    g = rows_ref[...]
    acc += jnp.maximum(g[0], 0.0) + jnp.maximum(g[1], 0.0)
    acc += jnp.maximum(g[2], 0.0)
    out = jnp.dot(acc, wout_ref[...], preferred_element_type=jnp.float32)
    out = jnp.maximum(out + b_ref[...], 0.0)
    o_ref[...] = jnp.where(sp_ref[...] != 0, out, 0.0)


def _tc_fused(rows3, mv, sd, vol, sp, actions, side_table, pmat, w_hex, w_out,
              b2):
    n_blocks = B // BB
    na = actions.shape[0]
    return pl.pallas_call(
        _fused_body,
        grid=(n_blocks,),
        in_specs=[
            pl.BlockSpec((NUM_SC_TABLES, BB, D), lambda i: (0, i, 0)),
            pl.BlockSpec((BB, 4), lambda i: (i, 0)),
            pl.BlockSpec((BB, 1), lambda i: (i, 0)),
            pl.BlockSpec((BB, NUM_VOLATILE_FIELDS), lambda i: (i, 0)),
            pl.BlockSpec((BB, 1), lambda i: (i, 0)),
            pl.BlockSpec((na, D), lambda i: (0, 0)),
            pl.BlockSpec((2, D), lambda i: (0, 0)),
            pl.BlockSpec((NUM_VOLATILE_FIELDS, HEX_FEATS), lambda i: (0, 0)),
            pl.BlockSpec((HEX_FEATS, D), lambda i: (0, 0)),
            pl.BlockSpec((D, D), lambda i: (0, 0)),
            pl.BlockSpec((1, D), lambda i: (0, 0)),
        ],
        out_specs=pl.BlockSpec((BB, D), lambda i: (i, 0)),
        out_shape=jax.ShapeDtypeStruct((B, D), jnp.float32),
    )(rows3, mv, sd, vol, sp, actions, side_table, pmat, w_hex, w_out, b2)


def kernel(species_idx, ability_idx, item_idx, side_idx, move_ids, volatiles,
           species_table, abilities_table, items_table, actions_table,
           side_table, W_hex, W_out, b_out):
    sp = species_idx.astype(jnp.int32)
    rows = _sc_gather(
        (species_table, abilities_table, items_table),
        (sp, ability_idx.astype(jnp.int32), item_idx.astype(jnp.int32)))
    rows3 = rows.reshape(NUM_SC_TABLES, B, D)
    actions_pad = jnp.zeros((1024, D), jnp.bfloat16).at[
        :actions_table.shape[0]].set(actions_table.astype(jnp.bfloat16))
    return _tc_fused(rows3, move_ids.astype(jnp.int32),
                     side_idx.astype(jnp.int32).reshape(B, 1),
                     volatiles.astype(jnp.int32), sp.reshape(B, 1),
                     actions_pad, side_table, jnp.asarray(_P_NP), W_hex,
                     W_out, b_out.reshape(1, D))


# speedup vs baseline: 1.1075x; 1.0802x over previous
import functools

import jax
import jax.numpy as jnp
import numpy as np
from jax import lax
from jax.experimental import pallas as pl
from jax.experimental.pallas import tpu as pltpu
from jax.experimental.pallas import tpu_sc as plsc

B = 4096
D = 128
NUM_SC_TABLES = 3
NC = 2
NS = 16
NW = NC * NS
SEG = B // NW
BB = 1024
HEX_BITS = 16
NUM_VOLATILE_FIELDS = 9
HEX_FEATS = NUM_VOLATILE_FIELDS * HEX_BITS

_P_NP = np.zeros((NUM_VOLATILE_FIELDS, HEX_FEATS), np.float32)
for _f in range(NUM_VOLATILE_FIELDS):
    for _k in range(HEX_BITS):
        _P_NP[_f, HEX_BITS * _f + _k] = 2.0 ** (-_k)


def _sc_gather(tables, idxs):
    mesh = plsc.VectorSubcoreMesh(core_axis_name="c", subcore_axis_name="s")
    n = NUM_SC_TABLES

    @functools.partial(
        pl.kernel,
        out_type=jax.ShapeDtypeStruct((n * B, D), jnp.float32),
        mesh=mesh,
        scratch_types=(
            [pltpu.VMEM((SEG,), jnp.int32) for _ in range(n)]
            + [pltpu.VMEM((SEG, D), jnp.float32) for _ in range(n)]
            + [pltpu.SemaphoreType.DMA for _ in range(3 * n)]
        ),
    )
    def gather_kernel(t0, t1, t2, i0, i1, i2, out_hbm, *scratch):
        tabs = (t0, t1, t2)
        idx_hbm = (i0, i1, i2)
        ib = scratch[:n]
        rb = scratch[n:2 * n]
        sis = scratch[2 * n:3 * n]
        sgs = scratch[3 * n:4 * n]
        sws = scratch[4 * n:5 * n]
        wid = lax.axis_index("s") * NC + lax.axis_index("c")
        base = wid * SEG

        icp = [
            pltpu.async_copy(idx_hbm[k].at[pl.ds(base, SEG)], ib[k], sis[k])
            for k in range(n)
        ]
        gcp = [None] * n
        wcp = [None] * n
        for k in range(n):
            icp[k].wait()
            gcp[k] = pltpu.async_copy(tabs[k].at[ib[k]], rb[k], sgs[k])
        for k in range(n):
            gcp[k].wait()
            wcp[k] = pltpu.async_copy(
                rb[k], out_hbm.at[pl.ds(k * B + base, SEG)], sws[k])
        for k in range(n):
            wcp[k].wait()

    return gather_kernel(*tables, *idxs)


def _fused_body(rows_ref, mv_ref, sd_ref, vol_ref, sp_ref, act_ref, side_ref,
                p_ref, whex_ref, wout_ref, b_ref, o_ref):
    mv = mv_ref[...]
    na = act_ref.shape[0]
    cols = lax.broadcasted_iota(jnp.int32, (BB, na), 1)
    counts = (
        (mv[:, 0:1] == cols).astype(jnp.bfloat16)
        + (mv[:, 1:2] == cols).astype(jnp.bfloat16)
        + (mv[:, 2:3] == cols).astype(jnp.bfloat16)
        + (mv[:, 3:4] == cols).astype(jnp.bfloat16)
    )
    mv_sum = jnp.dot(counts, act_ref[...],
                     preferred_element_type=jnp.float32)
    acc = jnp.maximum(mv_sum * 0.25, 0.0)
    sd = sd_ref[...]
    acc += jnp.maximum(
        jnp.where(sd == 0, side_ref[0:1, :], side_ref[1:2, :]), 0.0)
    del p_ref
    v = vol_ref[...]
    k16 = lax.broadcasted_iota(jnp.int32, (1, HEX_BITS), 1)
    for f in range(NUM_VOLATILE_FIELDS):
        bits_f = (jnp.right_shift(v[:, f : f + 1], k16) & 1).astype(
            jnp.float32)
        acc += jnp.dot(bits_f,
                       whex_ref[pl.ds(f * HEX_BITS, HEX_BITS), :],
                       preferred_element_type=jnp.float32)
    g = rows_ref[...]
    acc += jnp.maximum(g[0], 0.0) + jnp.maximum(g[1], 0.0)
    acc += jnp.maximum(g[2], 0.0)
    out = jnp.dot(acc, wout_ref[...], preferred_element_type=jnp.float32)
    out = jnp.maximum(out + b_ref[...], 0.0)
    o_ref[...] = jnp.where(sp_ref[...] != 0, out, 0.0)


def _tc_fused(rows3, mv, sd, vol, sp, actions, side_table, pmat, w_hex, w_out,
              b2):
    n_blocks = B // BB
    na = actions.shape[0]
    return pl.pallas_call(
        _fused_body,
        grid=(n_blocks,),
        in_specs=[
            pl.BlockSpec((NUM_SC_TABLES, BB, D), lambda i: (0, i, 0)),
            pl.BlockSpec((BB, 4), lambda i: (i, 0)),
            pl.BlockSpec((BB, 1), lambda i: (i, 0)),
            pl.BlockSpec((BB, NUM_VOLATILE_FIELDS), lambda i: (i, 0)),
            pl.BlockSpec((BB, 1), lambda i: (i, 0)),
            pl.BlockSpec((na, D), lambda i: (0, 0)),
            pl.BlockSpec((2, D), lambda i: (0, 0)),
            pl.BlockSpec((NUM_VOLATILE_FIELDS, HEX_FEATS), lambda i: (0, 0)),
            pl.BlockSpec((HEX_FEATS, D), lambda i: (0, 0)),
            pl.BlockSpec((D, D), lambda i: (0, 0)),
            pl.BlockSpec((1, D), lambda i: (0, 0)),
        ],
        out_specs=pl.BlockSpec((BB, D), lambda i: (i, 0)),
        out_shape=jax.ShapeDtypeStruct((B, D), jnp.float32),
    )(rows3, mv, sd, vol, sp, actions, side_table, pmat, w_hex, w_out, b2)


def kernel(species_idx, ability_idx, item_idx, side_idx, move_ids, volatiles,
           species_table, abilities_table, items_table, actions_table,
           side_table, W_hex, W_out, b_out):
    sp = species_idx.astype(jnp.int32)
    rows = _sc_gather(
        (species_table, abilities_table, items_table),
        (sp, ability_idx.astype(jnp.int32), item_idx.astype(jnp.int32)))
    rows3 = rows.reshape(NUM_SC_TABLES, B, D)
    actions_pad = jnp.zeros((1024, D), jnp.bfloat16).at[
        :actions_table.shape[0]].set(actions_table.astype(jnp.bfloat16))
    return _tc_fused(rows3, move_ids.astype(jnp.int32),
                     side_idx.astype(jnp.int32).reshape(B, 1),
                     volatiles.astype(jnp.int32), sp.reshape(B, 1),
                     actions_pad, side_table, jnp.asarray(_P_NP), W_hex,
                     W_out, b_out.reshape(1, D))


# speedup vs baseline: 1.1284x; 1.0189x over previous
import functools

import jax
import jax.numpy as jnp
from jax import lax
from jax.experimental import pallas as pl
from jax.experimental.pallas import tpu as pltpu
from jax.experimental.pallas import tpu_sc as plsc

B = 4096
D = 128
NUM_SC_TABLES = 3
NC = 2
NS = 16
NW = NC * NS
SEG = B // NW
BB = 1024
HEX_BITS = 16
NUM_VOLATILE_FIELDS = 9
HEX_FEATS = NUM_VOLATILE_FIELDS * HEX_BITS

def _sc_gather(tables, idxs):
    mesh = plsc.VectorSubcoreMesh(core_axis_name="c", subcore_axis_name="s")
    n = NUM_SC_TABLES

    @functools.partial(
        pl.kernel,
        out_type=jax.ShapeDtypeStruct((n * B, D), jnp.float32),
        mesh=mesh,
        scratch_types=(
            [pltpu.VMEM((SEG,), jnp.int32) for _ in range(n)]
            + [pltpu.VMEM((SEG, D), jnp.float32) for _ in range(n)]
            + [pltpu.SemaphoreType.DMA for _ in range(3 * n)]
        ),
    )
    def gather_kernel(t0, t1, t2, i0, i1, i2, out_hbm, *scratch):
        tabs = (t0, t1, t2)
        idx_hbm = (i0, i1, i2)
        ib = scratch[:n]
        rb = scratch[n:2 * n]
        sis = scratch[2 * n:3 * n]
        sgs = scratch[3 * n:4 * n]
        sws = scratch[4 * n:5 * n]
        wid = lax.axis_index("s") * NC + lax.axis_index("c")
        base = wid * SEG

        icp = [
            pltpu.async_copy(idx_hbm[k].at[pl.ds(base, SEG)], ib[k], sis[k])
            for k in range(n)
        ]
        gcp = [None] * n
        wcp = [None] * n
        for k in range(n):
            icp[k].wait()
            gcp[k] = pltpu.async_copy(tabs[k].at[ib[k]], rb[k], sgs[k])
        for k in range(n):
            gcp[k].wait()
            wcp[k] = pltpu.async_copy(
                rb[k], out_hbm.at[pl.ds(k * B + base, SEG)], sws[k])
        for k in range(n):
            wcp[k].wait()

    return gather_kernel(*tables, *idxs)


def _fused_body(rows_ref, mv_ref, sd_ref, vol_ref, sp_ref, act_ref, side_ref,
                whex_ref, wout_ref, b_ref, o_ref):
    mv = mv_ref[...]
    na = act_ref.shape[0]
    cols = lax.broadcasted_iota(jnp.int32, (BB, na), 1)
    counts = (
        (mv[:, 0:1] == cols).astype(jnp.bfloat16)
        + (mv[:, 1:2] == cols).astype(jnp.bfloat16)
        + (mv[:, 2:3] == cols).astype(jnp.bfloat16)
        + (mv[:, 3:4] == cols).astype(jnp.bfloat16)
    )
    mv_sum = jnp.dot(counts, act_ref[...],
                     preferred_element_type=jnp.float32)
    acc = jnp.maximum(mv_sum * 0.25, 0.0)
    sd = sd_ref[...]
    acc += jnp.maximum(
        jnp.where(sd == 0, side_ref[0:1, :], side_ref[1:2, :]), 0.0)
    v = vol_ref[...]
    k16 = lax.broadcasted_iota(jnp.int32, (1, HEX_BITS), 1)
    for f in range(NUM_VOLATILE_FIELDS):
        bits_f = (jnp.right_shift(v[:, f : f + 1], k16) & 1).astype(
            jnp.float32)
        acc += jnp.dot(bits_f,
                       whex_ref[pl.ds(f * HEX_BITS, HEX_BITS), :],
                       preferred_element_type=jnp.float32)
    g = rows_ref[...]
    acc += jnp.maximum(g[0], 0.0) + jnp.maximum(g[1], 0.0)
    acc += jnp.maximum(g[2], 0.0)
    out = jnp.dot(acc, wout_ref[...], preferred_element_type=jnp.float32)
    out = jnp.maximum(out + b_ref[...], 0.0)
    o_ref[...] = jnp.where(sp_ref[...] != 0, out, 0.0)


def _tc_fused(rows3, mv, sd, vol, sp, actions, side_table, w_hex, w_out, b2):
    n_blocks = B // BB
    na = actions.shape[0]
    return pl.pallas_call(
        _fused_body,
        grid=(n_blocks,),
        in_specs=[
            pl.BlockSpec((NUM_SC_TABLES, BB, D), lambda i: (0, i, 0)),
            pl.BlockSpec((BB, 4), lambda i: (i, 0)),
            pl.BlockSpec((BB, 1), lambda i: (i, 0)),
            pl.BlockSpec((BB, NUM_VOLATILE_FIELDS), lambda i: (i, 0)),
            pl.BlockSpec((BB, 1), lambda i: (i, 0)),
            pl.BlockSpec((na, D), lambda i: (0, 0)),
            pl.BlockSpec((2, D), lambda i: (0, 0)),
            pl.BlockSpec((HEX_FEATS, D), lambda i: (0, 0)),
            pl.BlockSpec((D, D), lambda i: (0, 0)),
            pl.BlockSpec((1, D), lambda i: (0, 0)),
        ],
        out_specs=pl.BlockSpec((BB, D), lambda i: (i, 0)),
        out_shape=jax.ShapeDtypeStruct((B, D), jnp.float32),
    )(rows3, mv, sd, vol, sp, actions, side_table, w_hex, w_out, b2)


def kernel(species_idx, ability_idx, item_idx, side_idx, move_ids, volatiles,
           species_table, abilities_table, items_table, actions_table,
           side_table, W_hex, W_out, b_out):
    sp = species_idx.astype(jnp.int32)
    rows = _sc_gather(
        (species_table, abilities_table, items_table),
        (sp, ability_idx.astype(jnp.int32), item_idx.astype(jnp.int32)))
    rows3 = rows.reshape(NUM_SC_TABLES, B, D)
    actions_pad = jnp.zeros((1024, D), jnp.bfloat16).at[
        :actions_table.shape[0]].set(actions_table.astype(jnp.bfloat16))
    return _tc_fused(rows3, move_ids.astype(jnp.int32),
                     side_idx.astype(jnp.int32).reshape(B, 1),
                     volatiles.astype(jnp.int32), sp.reshape(B, 1),
                     actions_pad, side_table, W_hex, W_out,
                     b_out.reshape(1, D))
